# baseline (speedup 1.0000x reference)
import functools

import numpy as np
import jax
import jax.numpy as jnp
from jax import lax
from jax.experimental import pallas as pl
from jax.experimental.pallas import tpu as pltpu
from jax.experimental.pallas import tpu_sc as plsc

VOCAB = 50000
COMP = 38500
TABLE = 500009
DIM = 32
HEADS = 4
ORDERS = (2, 3)
B, S = 4, 4096
N = B * S
NTAB = HEADS * len(ORDERS)

NC, NS, LANES = 2, 16, 16
NW = NC * NS
CHUNK = N // NW
GCHUNK = 128
NG = CHUNK // GCHUNK
VPG = GCHUNK // LANES


def _hash_consts():
    out = []
    for n in ORDERS:
        rng = np.random.RandomState(42 + n)
        coeffs = (rng.randint(1, 2**31 - 1, size=(HEADS, n)).astype(np.int64)) | 1
        seeds = rng.randint(0, 2**31 - 1, size=(HEADS,)).astype(np.int64)
        for h in range(HEADS):
            cm = [int(c % TABLE) for c in coeffs[h]]
            out.append((n,
                        [(c // 1024, c % 1024) for c in cm],
                        int(seeds[h] % TABLE)))
    return out


_TABLES = _hash_consts()

_mesh = plsc.VectorSubcoreMesh(core_axis_name="c", subcore_axis_name="s",
                               num_cores=NC, num_subcores=NS)


@functools.partial(
    pl.kernel,
    out_type=jax.ShapeDtypeStruct((N, NTAB * DIM), jnp.float32),
    mesh=_mesh,
    compiler_params=pltpu.CompilerParams(use_tc_tiling_on_sc=False),
    scratch_types=(
        [pltpu.VMEM((CHUNK,), jnp.int32) for _ in range(3)]
        + [pltpu.VMEM((NG, GCHUNK), jnp.int32),
           pltpu.VMEM((CHUNK, DIM), jnp.float32),
           pltpu.SemaphoreType.DMA]
    ),
)
def _engram_kernel(t0_hbm, t1_hbm, t2_hbm,
                   tab0, tab1, tab2, tab3, tab4, tab5, tab6, tab7,
                   out_hbm, g0_v, g1_v, g2_v, idx_v, rows_v, sem):
    tables = (tab0, tab1, tab2, tab3, tab4, tab5, tab6, tab7)
    g_bufs = (g0_v, g1_v, g2_v)
    wid = lax.axis_index("s") * NC + lax.axis_index("c")
    base = wid * CHUNK

    pltpu.sync_copy(t0_hbm.at[pl.ds(base, CHUNK)], g0_v)
    pltpu.sync_copy(t1_hbm.at[pl.ds(base, CHUNK)], g1_v)
    pltpu.sync_copy(t2_hbm.at[pl.ds(base, CHUNK)], g2_v)

    comp_c = jnp.int32(COMP)
    table_c = jnp.int32(TABLE)
    vpg_c = jnp.int32(VPG)

    def compress_body(i, carry):
        for gb in g_bufs:
            gb[pl.ds(i * LANES, LANES)] = lax.rem(gb[pl.ds(i * LANES, LANES)],
                                                  comp_c)
        return carry
    lax.fori_loop(jnp.int32(0), jnp.int32(CHUNK // LANES), compress_body,
                  jnp.int32(0))

    for k, (n, coefs, seed) in enumerate(_TABLES):
        def hash_body(i, carry, n=n, coefs=coefs, seed=seed):
            acc = jnp.full((LANES,), seed, dtype=jnp.int32)
            for p in range(n):
                g = g_bufs[n - 1 - p][pl.ds(i * LANES, LANES)]
                hi, lo = coefs[p]
                term = lax.rem(
                    lax.rem(g * jnp.int32(hi), table_c) * jnp.int32(1024)
                    + g * jnp.int32(lo), table_c)
                acc = acc + term
            idx_v[lax.div(i, vpg_c),
                  pl.ds(lax.rem(i, vpg_c) * LANES, LANES)] = lax.rem(acc,
                                                                    table_c)
            return carry
        lax.fori_loop(jnp.int32(0), jnp.int32(CHUNK // LANES), hash_body,
                      jnp.int32(0))

        def fire_body(j, carry, tab=tables[k]):
            pltpu.async_copy(tab.at[idx_v.at[j]],
                             rows_v.at[pl.ds(j * GCHUNK, GCHUNK)], sem)
            return carry
        lax.fori_loop(jnp.int32(0), jnp.int32(NG), fire_body, jnp.int32(0))

        def drain_body(j, carry, tab=tables[k]):
            pltpu.make_async_copy(tab.at[idx_v.at[j]],
                                  rows_v.at[pl.ds(j * GCHUNK, GCHUNK)],
                                  sem).wait()
            return carry
        lax.fori_loop(jnp.int32(0), jnp.int32(NG), drain_body, jnp.int32(0))

        pltpu.sync_copy(rows_v,
                        out_hbm.at[pl.ds(base, CHUNK), pl.ds(k * DIM, DIM)])


def kernel(token_ids, emb_n2_h0, emb_n2_h1, emb_n2_h2, emb_n2_h3,
           emb_n3_h0, emb_n3_h1, emb_n3_h2, emb_n3_h3):
    tok = token_ids.astype(jnp.int32)
    zeros = jnp.zeros((B, 1), dtype=jnp.int32)
    t1 = jnp.concatenate([zeros, tok[:, :-1]], axis=1)
    t2 = jnp.concatenate([zeros, zeros, tok[:, :-2]], axis=1)
    out = _engram_kernel(tok.reshape(N), t1.reshape(N), t2.reshape(N),
                         emb_n2_h0, emb_n2_h1, emb_n2_h2, emb_n2_h3,
                         emb_n3_h0, emb_n3_h1, emb_n3_h2, emb_n3_h3)
    return out.reshape(B, S, NTAB * DIM)


# speedup vs baseline: 1.0074x; 1.0074x over previous
import functools

import numpy as np
import jax
import jax.numpy as jnp
from jax import lax
from jax.experimental import pallas as pl
from jax.experimental.pallas import tpu as pltpu
from jax.experimental.pallas import tpu_sc as plsc

VOCAB = 50000
COMP = 38500
TABLE = 500009
DIM = 32
HEADS = 4
ORDERS = (2, 3)
B, S = 4, 4096
N = B * S
NTAB = HEADS * len(ORDERS)

NC, NS, LANES = 2, 16, 16
NW = NC * NS
CHUNK = N // NW
GCHUNK = 128
NG = CHUNK // GCHUNK
VPG = GCHUNK // LANES


def _hash_consts():
    out = []
    for n in ORDERS:
        rng = np.random.RandomState(42 + n)
        coeffs = (rng.randint(1, 2**31 - 1, size=(HEADS, n)).astype(np.int64)) | 1
        seeds = rng.randint(0, 2**31 - 1, size=(HEADS,)).astype(np.int64)
        for h in range(HEADS):
            cm = [int(c % TABLE) for c in coeffs[h]]
            out.append((n,
                        [(c // 1024, c % 1024) for c in cm],
                        int(seeds[h] % TABLE)))
    return out


_TABLES = _hash_consts()

_mesh = plsc.VectorSubcoreMesh(core_axis_name="c", subcore_axis_name="s",
                               num_cores=NC, num_subcores=NS)


@functools.partial(
    pl.kernel,
    out_type=jax.ShapeDtypeStruct((N, NTAB * DIM), jnp.float32),
    mesh=_mesh,
    compiler_params=pltpu.CompilerParams(use_tc_tiling_on_sc=False),
    scratch_types=(
        [pltpu.VMEM((CHUNK,), jnp.int32) for _ in range(3)]
        + [pltpu.VMEM((NG, GCHUNK), jnp.int32) for _ in range(2)]
        + [pltpu.VMEM((CHUNK, DIM), jnp.float32) for _ in range(2)]
        + [pltpu.SemaphoreType.DMA, pltpu.SemaphoreType.DMA,
           pltpu.SemaphoreType.DMA]
    ),
)
def _engram_kernel(t0_hbm, t1_hbm, t2_hbm,
                   tab0, tab1, tab2, tab3, tab4, tab5, tab6, tab7,
                   out_hbm, g0_v, g1_v, g2_v,
                   idx_a, idx_b, rows_a, rows_b, gsem_a, gsem_b, wsem):
    tables = (tab0, tab1, tab2, tab3, tab4, tab5, tab6, tab7)
    g_bufs = (g0_v, g1_v, g2_v)
    idxs = (idx_a, idx_b)
    rows = (rows_a, rows_b)
    gsems = (gsem_a, gsem_b)
    wid = lax.axis_index("s") * NC + lax.axis_index("c")
    base = wid * CHUNK

    pltpu.sync_copy(t0_hbm.at[pl.ds(base, CHUNK)], g0_v)
    pltpu.sync_copy(t1_hbm.at[pl.ds(base, CHUNK)], g1_v)
    pltpu.sync_copy(t2_hbm.at[pl.ds(base, CHUNK)], g2_v)

    comp_c = jnp.int32(COMP)
    table_c = jnp.int32(TABLE)
    vpg_c = jnp.int32(VPG)

    def compress_body(i, carry):
        for gb in g_bufs:
            gb[pl.ds(i * LANES, LANES)] = lax.rem(gb[pl.ds(i * LANES, LANES)],
                                                  comp_c)
        return carry
    lax.fori_loop(jnp.int32(0), jnp.int32(CHUNK // LANES), compress_body,
                  jnp.int32(0))

    def fire_gathers(k, p):
        def fire_body(j, carry, tab=tables[k], idx_v=idxs[p], rv=rows[p],
                      sem=gsems[p]):
            pltpu.async_copy(tab.at[idx_v.at[j]],
                             rv.at[pl.ds(j * GCHUNK, GCHUNK)], sem)
            return carry
        lax.fori_loop(jnp.int32(0), jnp.int32(NG), fire_body, jnp.int32(0))

    def drain_gathers(k, p):
        def drain_body(j, carry, tab=tables[k], idx_v=idxs[p], rv=rows[p],
                       sem=gsems[p]):
            pltpu.make_async_copy(tab.at[idx_v.at[j]],
                                  rv.at[pl.ds(j * GCHUNK, GCHUNK)],
                                  sem).wait()
            return carry
        lax.fori_loop(jnp.int32(0), jnp.int32(NG), drain_body, jnp.int32(0))

    def write_desc(k, p):
        return pltpu.make_async_copy(
            rows[p], out_hbm.at[pl.ds(base, CHUNK), pl.ds(k * DIM, DIM)], wsem)

    for k, (n, coefs, seed) in enumerate(_TABLES):
        p = k % 2

        def hash_body(i, carry, n=n, coefs=coefs, seed=seed, idx_v=idxs[p]):
            acc = jnp.full((LANES,), seed, dtype=jnp.int32)
            for q in range(n):
                g = g_bufs[n - 1 - q][pl.ds(i * LANES, LANES)]
                hi, lo = coefs[q]
                term = lax.rem(
                    lax.rem(g * jnp.int32(hi), table_c) * jnp.int32(1024)
                    + g * jnp.int32(lo), table_c)
                acc = acc + term
            idx_v[lax.div(i, vpg_c),
                  pl.ds(lax.rem(i, vpg_c) * LANES, LANES)] = lax.rem(acc,
                                                                    table_c)
            return carry
        lax.fori_loop(jnp.int32(0), jnp.int32(CHUNK // LANES), hash_body,
                      jnp.int32(0))

        if k >= 2:
            write_desc(k - 2, p).wait()

        fire_gathers(k, p)

        if k >= 1:
            drain_gathers(k - 1, 1 - p)
            write_desc(k - 1, 1 - p).start()

    last = NTAB - 1
    lp = last % 2
    drain_gathers(last, lp)
    write_desc(last, lp).start()
    write_desc(last - 1, 1 - lp).wait()
    write_desc(last, lp).wait()


def kernel(token_ids, emb_n2_h0, emb_n2_h1, emb_n2_h2, emb_n2_h3,
           emb_n3_h0, emb_n3_h1, emb_n3_h2, emb_n3_h3):
    tok = token_ids.astype(jnp.int32)
    zeros = jnp.zeros((B, 1), dtype=jnp.int32)
    t1 = jnp.concatenate([zeros, tok[:, :-1]], axis=1)
    t2 = jnp.concatenate([zeros, zeros, tok[:, :-2]], axis=1)
    out = _engram_kernel(tok.reshape(N), t1.reshape(N), t2.reshape(N),
                         emb_n2_h0, emb_n2_h1, emb_n2_h2, emb_n2_h3,
                         emb_n3_h0, emb_n3_h1, emb_n3_h2, emb_n3_h3)
    return out.reshape(B, S, NTAB * DIM)
